# baseline (speedup 1.0000x reference)
import functools

import jax
import jax.numpy as jnp
from jax.experimental import pallas as pl

_HID = 64
_EPS = 1e-10


def _edge_topk_body(es_ref, g_ref, idx_ref, *, n, k):
    es = es_ref[0]
    g = g_ref[0]
    li = jax.lax.broadcasted_iota(jnp.int32, (n, n), 1)
    si = jax.lax.broadcasted_iota(jnp.int32, (n, n), 0)
    vals = jnp.where(li == si, -jnp.inf, g - es)
    for t in range(k):
        m = jnp.max(vals, axis=1, keepdims=True)
        cand = jnp.where(vals == m, li, n)
        j = jnp.min(cand, axis=1, keepdims=True)
        idx_ref[0, :, t : t + 1] = j
        vals = jnp.where(li == j, -jnp.inf, vals)


def _edge_topk(es, g, k):
    b, n, _ = es.shape
    return pl.pallas_call(
        functools.partial(_edge_topk_body, n=n, k=k),
        grid=(b,),
        in_specs=[
            pl.BlockSpec((1, n, n), lambda i: (i, 0, 0)),
            pl.BlockSpec((1, n, n), lambda i: (i, 0, 0)),
        ],
        out_specs=pl.BlockSpec((1, n, k), lambda i: (i, 0, 0)),
        out_shape=jax.ShapeDtypeStruct((b, n, k), jnp.int32),
    )(es, g)


def _node_order_body(vc_ref, vr_ref, idx_ref, *, n, m):
    vc = vc_ref[0]
    vr = vr_ref[0]
    si = jax.lax.broadcasted_iota(jnp.int32, (n, n), 0)
    li = jax.lax.broadcasted_iota(jnp.int32, (n, n), 1)
    beats = (vc > vr) | ((vc == vr) & (si < li))
    rank = jnp.sum(beats.astype(jnp.int32), axis=0, keepdims=True)
    ri = jax.lax.broadcasted_iota(jnp.int32, (m, n), 0)
    li2 = jax.lax.broadcasted_iota(jnp.int32, (m, n), 1)
    idx = jnp.sum(jnp.where(rank == ri, li2, 0), axis=1, keepdims=True)
    idx_ref[0] = idx


def _node_order(v, m):
    b, n = v.shape
    vc = v.reshape(b, n, 1)
    vr = v.reshape(b, 1, n)
    return pl.pallas_call(
        functools.partial(_node_order_body, n=n, m=m),
        grid=(b,),
        in_specs=[
            pl.BlockSpec((1, n, 1), lambda i: (i, 0, 0)),
            pl.BlockSpec((1, 1, n), lambda i: (i, 0, 0)),
        ],
        out_specs=pl.BlockSpec((1, m, 1), lambda i: (i, 0, 0)),
        out_shape=jax.ShapeDtypeStruct((b, m, 1), jnp.int32),
    )(vc, vr)


def _dotT(a, w, prec):
    return jax.lax.dot_general(
        a, w, (((1,), (1,)), ((), ())),
        precision=prec, preferred_element_type=jnp.float32)


def _stat(xT, T):
    mu = jnp.mean(xT, axis=1, keepdims=True)
    var = jnp.sum((xT - mu) ** 2, axis=1, keepdims=True) * (1.0 / (T - 1))
    return mu, 1.0 / (jnp.sqrt(var) + _EPS)


def _gcn_body(xT_ref, xsT_ref, fc1w_ref, fc1b_ref, fc2w_ref, fc2b_ref,
              w0_ref, b0_ref, w1_ref, b1_ref, w2_ref, b2_ref,
              idx0_ref, idx1_ref, idx2_ref, n1_ref, n2_ref, out_ref,
              *, n0, T):
    f32 = jnp.float32
    hi = jax.lax.Precision.HIGHEST
    xT = xT_ref[0]
    xsT = xsT_ref[0]

    x1, x2 = _stat(xT, T)
    x3, x4 = _stat(xsT, T)
    h = jnp.concatenate([x1, x2, x3, x4], axis=1)
    mu = jnp.mean(h, axis=0, keepdims=True)
    sd = jnp.sqrt(jnp.sum((h - mu) ** 2, axis=0, keepdims=True)
                  * (1.0 / (n0 - 1)))
    h = (h - mu) / (sd + _EPS)

    h = _dotT(h, fc1w_ref[...], hi) + fc1b_ref[...]

    def gcn(h, idx, w, b, n, k):
        hw = _dotT(h, w, hi)
        li = jax.lax.broadcasted_iota(jnp.int32, (n, n), 1)
        m = jnp.zeros((n, n), f32)
        for t in range(k):
            m = m + (idx[:, t : t + 1] == li).astype(f32)
        agg = (jnp.dot(m, hw, precision=hi,
                       preferred_element_type=f32) + hw) * (1.0 / (k + 1)) + b
        return jnp.where(agg >= 0, agg, 0.01 * agg)

    def gather_rows(h, nidx, m, n):
        li = jax.lax.broadcasted_iota(jnp.int32, (m, n), 1)
        p = (nidx == li).astype(f32)
        return jnp.dot(p, h, precision=hi, preferred_element_type=f32)

    h = gcn(h, idx0_ref[0], w0_ref[...], b0_ref[...], n0, 8)
    h = gather_rows(h, n1_ref[0], n0 // 2, n0)
    h = gcn(h, idx1_ref[0], w1_ref[...], b1_ref[...], n0 // 2, 4)
    h = gather_rows(h, n2_ref[0], n0 // 4, n0 // 2)
    h = gcn(h, idx2_ref[0], w2_ref[...], b2_ref[...], n0 // 4, 2)

    hh = _dotT(h, fc2w_ref[...], hi) + fc2b_ref[...]
    gram = jax.lax.dot_general(hh, hh, (((1,), (1,)), ((), ())),
                               precision=hi, preferred_element_type=f32)
    out_ref[0] = jax.nn.sigmoid(gram)


def _gcn_run(xT, xsT, fc1_w, fc1_b, fc2_w, fc2_b, gws, gbs,
             idx0, idx1, idx2, n1, n2):
    b, n0, T = xT.shape
    nf = n0 // 4
    full = lambda *shape: pl.BlockSpec(shape, lambda i: tuple(0 for _ in shape))
    bat = lambda *shape: pl.BlockSpec((1,) + shape,
                                      lambda i: (i,) + tuple(0 for _ in shape))
    args = (
        xT, xsT,
        fc1_w, fc1_b.reshape(1, _HID),
        jnp.zeros((128, _HID), jnp.float32).at[0].set(fc2_w[0]),
        jnp.zeros((1, 128), jnp.float32).at[0, 0].set(fc2_b[0]),
        gws[0], gbs[0].reshape(1, _HID),
        gws[1], gbs[1].reshape(1, _HID),
        gws[2], gbs[2].reshape(1, _HID),
        idx0, idx1, idx2, n1, n2,
    )
    in_specs = [
        bat(n0, T), bat(n0, T),
        full(_HID, 4), full(1, _HID), full(128, _HID), full(1, 128),
        full(_HID, _HID), full(1, _HID),
        full(_HID, _HID), full(1, _HID),
        full(_HID, _HID), full(1, _HID),
        bat(n0, 8), bat(n0 // 2, 4), bat(nf, 2),
        bat(n0 // 2, 1), bat(nf, 1),
    ]
    return pl.pallas_call(
        functools.partial(_gcn_body, n0=n0, T=T),
        grid=(b,),
        in_specs=in_specs,
        out_specs=pl.BlockSpec((1, nf, nf), lambda i: (i, 0, 0)),
        out_shape=jax.ShapeDtypeStruct((b, nf, nf), jnp.float32),
    )(*args)


@functools.lru_cache(maxsize=None)
def _gumbel_noise(b, n0):
    with jax.ensure_compile_time_eval():
        key = jax.random.key(42)
        key, k1 = jax.random.split(key)
        ge0 = jax.random.gumbel(k1, (b, n0, n0), jnp.float32)
        key, k2 = jax.random.split(key)
        gn0 = jax.random.gumbel(k2, (b, n0), jnp.float32)
        key, k1 = jax.random.split(key)
        ge1 = jax.random.gumbel(k1, (b, n0 // 2, n0 // 2), jnp.float32)
        key, k2 = jax.random.split(key)
        gn1 = jax.random.gumbel(k2, (b, n0 // 2), jnp.float32)
        key, k1 = jax.random.split(key)
        ge2 = jax.random.gumbel(k1, (b, n0 // 4, n0 // 4), jnp.float32)
    return ge0, gn0, ge1, gn1, ge2


def kernel(x, edge_score, node_score, fc1_w, fc1_b, fc2_w, fc2_b,
           g0_w, g0_b, g1_w, g1_b, g2_w, g2_b):
    b, T, n0 = x.shape
    ge0, gn0, ge1, gn1, ge2 = _gumbel_noise(b, n0)

    vn0 = jax.nn.log_softmax(-node_score, axis=-1) + gn0
    n1 = _node_order(vn0, n0 // 2)
    n1f = n1[..., 0]
    ns1 = jnp.take_along_axis(node_score, n1f, axis=1)
    es1 = jnp.take_along_axis(edge_score, n1f[:, :, None], axis=1)
    es1 = jnp.take_along_axis(es1, n1f[:, None, :], axis=2)
    vn1 = jax.nn.log_softmax(-ns1, axis=-1) + gn1
    n2 = _node_order(vn1, n0 // 4)
    n2f = n2[..., 0]
    es2 = jnp.take_along_axis(es1, n2f[:, :, None], axis=1)
    es2 = jnp.take_along_axis(es2, n2f[:, None, :], axis=2)
    selected = jnp.take_along_axis(n1f, n2f, axis=1)

    idx0 = _edge_topk(edge_score, ge0, 8)
    idx1 = _edge_topk(es1, ge1, 4)
    idx2 = _edge_topk(es2, ge2, 2)

    ord_ = jnp.argsort(x, axis=-1)
    bi = jnp.arange(b, dtype=jnp.int32)[:, None, None]
    ti = jnp.arange(T, dtype=jnp.int32)[None, :, None]
    ri = jnp.broadcast_to(jnp.arange(n0, dtype=jnp.int32), (b, T, n0))
    xs = jnp.zeros((b, T, n0), jnp.int32).at[bi, ti, ord_].set(
        ri, unique_indices=True, indices_are_sorted=False).astype(x.dtype)
    xT = jnp.swapaxes(x, 1, 2)
    xsT = jnp.swapaxes(xs, 1, 2)

    out = _gcn_run(xT, xsT, fc1_w, fc1_b, fc2_w, fc2_b,
                   [g0_w, g1_w, g2_w], [g0_b, g1_b, g2_b],
                   idx0, idx1, idx2, n1, n2)
    return out, selected


# speedup vs baseline: 1.0031x; 1.0031x over previous
import functools

import jax
import jax.numpy as jnp
from jax.experimental import pallas as pl

_HID = 64
_EPS = 1e-10


def _edge_topk_body(es_ref, g_ref, idx_ref, *, n, k):
    es = es_ref[0]
    g = g_ref[0]
    li = jax.lax.broadcasted_iota(jnp.int32, (n, n), 1)
    si = jax.lax.broadcasted_iota(jnp.int32, (n, n), 0)
    vals = jnp.where(li == si, -jnp.inf, g - es)
    for t in range(k):
        m = jnp.max(vals, axis=1, keepdims=True)
        cand = jnp.where(vals == m, li, n)
        j = jnp.min(cand, axis=1, keepdims=True)
        idx_ref[0, :, t : t + 1] = j
        vals = jnp.where(li == j, -jnp.inf, vals)


def _edge_topk(es, g, k):
    b, n, _ = es.shape
    return pl.pallas_call(
        functools.partial(_edge_topk_body, n=n, k=k),
        grid=(b,),
        in_specs=[
            pl.BlockSpec((1, n, n), lambda i: (i, 0, 0)),
            pl.BlockSpec((1, n, n), lambda i: (i, 0, 0)),
        ],
        out_specs=pl.BlockSpec((1, n, k), lambda i: (i, 0, 0)),
        out_shape=jax.ShapeDtypeStruct((b, n, k), jnp.int32),
    )(es, g)


def _node_order_body(vc_ref, vr_ref, idx_ref, *, n, m):
    vc = vc_ref[0]
    vr = vr_ref[0]
    si = jax.lax.broadcasted_iota(jnp.int32, (n, n), 0)
    li = jax.lax.broadcasted_iota(jnp.int32, (n, n), 1)
    beats = (vc > vr) | ((vc == vr) & (si < li))
    rank = jnp.sum(beats.astype(jnp.int32), axis=0, keepdims=True)
    ri = jax.lax.broadcasted_iota(jnp.int32, (m, n), 0)
    li2 = jax.lax.broadcasted_iota(jnp.int32, (m, n), 1)
    idx = jnp.sum(jnp.where(rank == ri, li2, 0), axis=1, keepdims=True)
    idx_ref[0] = idx


def _node_order(v, m):
    b, n = v.shape
    vc = v.reshape(b, n, 1)
    vr = v.reshape(b, 1, n)
    return pl.pallas_call(
        functools.partial(_node_order_body, n=n, m=m),
        grid=(b,),
        in_specs=[
            pl.BlockSpec((1, n, 1), lambda i: (i, 0, 0)),
            pl.BlockSpec((1, 1, n), lambda i: (i, 0, 0)),
        ],
        out_specs=pl.BlockSpec((1, m, 1), lambda i: (i, 0, 0)),
        out_shape=jax.ShapeDtypeStruct((b, m, 1), jnp.int32),
    )(vc, vr)


def _dotT(a, w, prec):
    return jax.lax.dot_general(
        a, w, (((1,), (1,)), ((), ())),
        precision=prec, preferred_element_type=jnp.float32)


def _stat(xT, T):
    mu = jnp.mean(xT, axis=1, keepdims=True)
    var = jnp.sum((xT - mu) ** 2, axis=1, keepdims=True) * (1.0 / (T - 1))
    return mu, 1.0 / (jnp.sqrt(var) + _EPS)


def _gcn_body(xT_ref, xsT_ref, fc1w_ref, fc1b_ref, fc2w_ref, fc2b_ref,
              w0_ref, b0_ref, w1_ref, b1_ref, w2_ref, b2_ref,
              idx0_ref, idx1_ref, idx2_ref, n1_ref, n2_ref, out_ref,
              *, n0, T):
    f32 = jnp.float32
    hi = jax.lax.Precision.HIGHEST
    xT = xT_ref[0]
    xsT = xsT_ref[0]

    x1, x2 = _stat(xT, T)
    x3, x4 = _stat(xsT, T)
    h = jnp.concatenate([x1, x2, x3, x4], axis=1)
    mu = jnp.mean(h, axis=0, keepdims=True)
    sd = jnp.sqrt(jnp.sum((h - mu) ** 2, axis=0, keepdims=True)
                  * (1.0 / (n0 - 1)))
    h = (h - mu) / (sd + _EPS)

    h = _dotT(h, fc1w_ref[...], hi) + fc1b_ref[...]

    def gcn(h, idx, w, b, n, k):
        hw = _dotT(h, w, hi)
        li = jax.lax.broadcasted_iota(jnp.int32, (n, n), 1)
        m = jnp.zeros((n, n), f32)
        for t in range(k):
            m = m + (idx[:, t : t + 1] == li).astype(f32)
        agg = (jnp.dot(m, hw, precision=hi,
                       preferred_element_type=f32) + hw) * (1.0 / (k + 1)) + b
        return jnp.where(agg >= 0, agg, 0.01 * agg)

    def gather_rows(h, nidx, m, n):
        li = jax.lax.broadcasted_iota(jnp.int32, (m, n), 1)
        p = (nidx == li).astype(f32)
        return jnp.dot(p, h, precision=hi, preferred_element_type=f32)

    h = gcn(h, idx0_ref[0], w0_ref[...], b0_ref[...], n0, 8)
    h = gather_rows(h, n1_ref[0], n0 // 2, n0)
    h = gcn(h, idx1_ref[0], w1_ref[...], b1_ref[...], n0 // 2, 4)
    h = gather_rows(h, n2_ref[0], n0 // 4, n0 // 2)
    h = gcn(h, idx2_ref[0], w2_ref[...], b2_ref[...], n0 // 4, 2)

    hh = _dotT(h, fc2w_ref[...], hi) + fc2b_ref[...]
    gram = jax.lax.dot_general(hh, hh, (((1,), (1,)), ((), ())),
                               precision=hi, preferred_element_type=f32)
    out_ref[0] = jax.nn.sigmoid(gram)


def _gcn_run(xT, xsT, fc1_w, fc1_b, fc2_w, fc2_b, gws, gbs,
             idx0, idx1, idx2, n1, n2):
    b, n0, T = xT.shape
    nf = n0 // 4
    full = lambda *shape: pl.BlockSpec(shape, lambda i: tuple(0 for _ in shape))
    bat = lambda *shape: pl.BlockSpec((1,) + shape,
                                      lambda i: (i,) + tuple(0 for _ in shape))
    args = (
        xT, xsT,
        fc1_w, fc1_b.reshape(1, _HID),
        jnp.zeros((128, _HID), jnp.float32).at[0].set(fc2_w[0]),
        jnp.zeros((1, 128), jnp.float32).at[0, 0].set(fc2_b[0]),
        gws[0], gbs[0].reshape(1, _HID),
        gws[1], gbs[1].reshape(1, _HID),
        gws[2], gbs[2].reshape(1, _HID),
        idx0, idx1, idx2, n1, n2,
    )
    in_specs = [
        bat(n0, T), bat(n0, T),
        full(_HID, 4), full(1, _HID), full(128, _HID), full(1, 128),
        full(_HID, _HID), full(1, _HID),
        full(_HID, _HID), full(1, _HID),
        full(_HID, _HID), full(1, _HID),
        bat(n0, 8), bat(n0 // 2, 4), bat(nf, 2),
        bat(n0 // 2, 1), bat(nf, 1),
    ]
    return pl.pallas_call(
        functools.partial(_gcn_body, n0=n0, T=T),
        grid=(b,),
        in_specs=in_specs,
        out_specs=pl.BlockSpec((1, nf, nf), lambda i: (i, 0, 0)),
        out_shape=jax.ShapeDtypeStruct((b, nf, nf), jnp.float32),
    )(*args)


@functools.lru_cache(maxsize=None)
def _gumbel_noise(b, n0):
    with jax.ensure_compile_time_eval():
        key = jax.random.key(42)
        key, k1 = jax.random.split(key)
        ge0 = jax.random.gumbel(k1, (b, n0, n0), jnp.float32)
        key, k2 = jax.random.split(key)
        gn0 = jax.random.gumbel(k2, (b, n0), jnp.float32)
        key, k1 = jax.random.split(key)
        ge1 = jax.random.gumbel(k1, (b, n0 // 2, n0 // 2), jnp.float32)
        key, k2 = jax.random.split(key)
        gn1 = jax.random.gumbel(k2, (b, n0 // 2), jnp.float32)
        key, k1 = jax.random.split(key)
        ge2 = jax.random.gumbel(k1, (b, n0 // 4, n0 // 4), jnp.float32)
    return ge0, gn0, ge1, gn1, ge2


def kernel(x, edge_score, node_score, fc1_w, fc1_b, fc2_w, fc2_b,
           g0_w, g0_b, g1_w, g1_b, g2_w, g2_b):
    b, T, n0 = x.shape
    ge0, gn0, ge1, gn1, ge2 = _gumbel_noise(b, n0)

    vn0 = jax.nn.log_softmax(-node_score, axis=-1) + gn0
    n1 = _node_order(vn0, n0 // 2)
    n1f = n1[..., 0]
    ns1 = jnp.take_along_axis(node_score, n1f, axis=1)
    es1 = jnp.take_along_axis(edge_score, n1f[:, :, None], axis=1)
    es1 = jnp.take_along_axis(es1, n1f[:, None, :], axis=2)
    vn1 = jax.nn.log_softmax(-ns1, axis=-1) + gn1
    n2 = _node_order(vn1, n0 // 4)
    n2f = n2[..., 0]
    es2 = jnp.take_along_axis(es1, n2f[:, :, None], axis=1)
    es2 = jnp.take_along_axis(es2, n2f[:, None, :], axis=2)
    selected = jnp.take_along_axis(n1f, n2f, axis=1)

    idx0 = _edge_topk(edge_score, ge0, 8)
    idx1 = _edge_topk(es1, ge1, 4)
    idx2 = _edge_topk(es2, ge2, 2)

    ord_ = jnp.argsort(x, axis=-1)
    base = (jnp.arange(b * T, dtype=jnp.int32) * n0).reshape(b, T, 1)
    lin = (ord_ + base).reshape(-1)
    rvals = jnp.broadcast_to(
        jnp.arange(n0, dtype=x.dtype), (b, T, n0)).reshape(-1)
    xs = jnp.zeros((b * T * n0,), x.dtype).at[lin].set(
        rvals, unique_indices=True).reshape(b, T, n0)
    xT = jnp.swapaxes(x, 1, 2)
    xsT = jnp.swapaxes(xs, 1, 2)

    out = _gcn_run(xT, xsT, fc1_w, fc1_b, fc2_w, fc2_b,
                   [g0_w, g1_w, g2_w], [g0_b, g1_b, g2_b],
                   idx0, idx1, idx2, n1, n2)
    return out, selected


# speedup vs baseline: 8.5837x; 8.5568x over previous
import functools

import jax
import jax.numpy as jnp
from jax.experimental import pallas as pl
from jax.experimental.pallas import tpu as pltpu

_HID = 64
_EPS = 1e-10


def _edge_topk_body(es_ref, g_ref, idx_ref, *, n, k):
    es = es_ref[0]
    g = g_ref[0]
    li = jax.lax.broadcasted_iota(jnp.int32, (n, n), 1)
    si = jax.lax.broadcasted_iota(jnp.int32, (n, n), 0)
    vals = jnp.where(li == si, -jnp.inf, g - es)
    for t in range(k):
        m = jnp.max(vals, axis=1, keepdims=True)
        cand = jnp.where(vals == m, li, n)
        j = jnp.min(cand, axis=1, keepdims=True)
        idx_ref[0, :, t : t + 1] = j
        vals = jnp.where(li == j, -jnp.inf, vals)


def _edge_topk(es, g, k):
    b, n, _ = es.shape
    return pl.pallas_call(
        functools.partial(_edge_topk_body, n=n, k=k),
        grid=(b,),
        in_specs=[
            pl.BlockSpec((1, n, n), lambda i: (i, 0, 0)),
            pl.BlockSpec((1, n, n), lambda i: (i, 0, 0)),
        ],
        out_specs=pl.BlockSpec((1, n, k), lambda i: (i, 0, 0)),
        out_shape=jax.ShapeDtypeStruct((b, n, k), jnp.int32),
    )(es, g)


def _node_order_body(vc_ref, vr_ref, idx_ref, *, n, m):
    vc = vc_ref[0]
    vr = vr_ref[0]
    si = jax.lax.broadcasted_iota(jnp.int32, (n, n), 0)
    li = jax.lax.broadcasted_iota(jnp.int32, (n, n), 1)
    beats = (vc > vr) | ((vc == vr) & (si < li))
    rank = jnp.sum(beats.astype(jnp.int32), axis=0, keepdims=True)
    ri = jax.lax.broadcasted_iota(jnp.int32, (m, n), 0)
    li2 = jax.lax.broadcasted_iota(jnp.int32, (m, n), 1)
    idx = jnp.sum(jnp.where(rank == ri, li2, 0), axis=1, keepdims=True)
    idx_ref[0] = idx


def _node_order(v, m):
    b, n = v.shape
    vc = v.reshape(b, n, 1)
    vr = v.reshape(b, 1, n)
    return pl.pallas_call(
        functools.partial(_node_order_body, n=n, m=m),
        grid=(b,),
        in_specs=[
            pl.BlockSpec((1, n, 1), lambda i: (i, 0, 0)),
            pl.BlockSpec((1, 1, n), lambda i: (i, 0, 0)),
        ],
        out_specs=pl.BlockSpec((1, m, 1), lambda i: (i, 0, 0)),
        out_shape=jax.ShapeDtypeStruct((b, m, 1), jnp.int32),
    )(vc, vr)


def _dotT(a, w, prec):
    return jax.lax.dot_general(
        a, w, (((1,), (1,)), ((), ())),
        precision=prec, preferred_element_type=jnp.float32)


def _stat(xT, T):
    mu = jnp.mean(xT, axis=1, keepdims=True)
    var = jnp.sum((xT - mu) ** 2, axis=1, keepdims=True) * (1.0 / (T - 1))
    return mu, 1.0 / (jnp.sqrt(var) + _EPS)


def _gcn_body(xT_ref, xsT_ref, fc1w_ref, fc1b_ref, fc2w_ref, fc2b_ref,
              w0_ref, b0_ref, w1_ref, b1_ref, w2_ref, b2_ref,
              idx0_ref, idx1_ref, idx2_ref, n1_ref, n2_ref, out_ref,
              *, n0, T):
    f32 = jnp.float32
    hi = jax.lax.Precision.HIGHEST
    xT = xT_ref[0]
    xsT = xsT_ref[0]

    x1, x2 = _stat(xT, T)
    x3, x4 = _stat(xsT, T)
    h = jnp.concatenate([x1, x2, x3, x4], axis=1)
    mu = jnp.mean(h, axis=0, keepdims=True)
    sd = jnp.sqrt(jnp.sum((h - mu) ** 2, axis=0, keepdims=True)
                  * (1.0 / (n0 - 1)))
    h = (h - mu) / (sd + _EPS)

    h = _dotT(h, fc1w_ref[...], hi) + fc1b_ref[...]

    def gcn(h, idx, w, b, n, k):
        hw = _dotT(h, w, hi)
        li = jax.lax.broadcasted_iota(jnp.int32, (n, n), 1)
        m = jnp.zeros((n, n), f32)
        for t in range(k):
            m = m + (idx[:, t : t + 1] == li).astype(f32)
        agg = (jnp.dot(m, hw, precision=hi,
                       preferred_element_type=f32) + hw) * (1.0 / (k + 1)) + b
        return jnp.where(agg >= 0, agg, 0.01 * agg)

    def gather_rows(h, nidx, m, n):
        li = jax.lax.broadcasted_iota(jnp.int32, (m, n), 1)
        p = (nidx == li).astype(f32)
        return jnp.dot(p, h, precision=hi, preferred_element_type=f32)

    h = gcn(h, idx0_ref[0], w0_ref[...], b0_ref[...], n0, 8)
    h = gather_rows(h, n1_ref[0], n0 // 2, n0)
    h = gcn(h, idx1_ref[0], w1_ref[...], b1_ref[...], n0 // 2, 4)
    h = gather_rows(h, n2_ref[0], n0 // 4, n0 // 2)
    h = gcn(h, idx2_ref[0], w2_ref[...], b2_ref[...], n0 // 4, 2)

    hh = _dotT(h, fc2w_ref[...], hi) + fc2b_ref[...]
    gram = jax.lax.dot_general(hh, hh, (((1,), (1,)), ((), ())),
                               precision=hi, preferred_element_type=f32)
    out_ref[0] = jax.nn.sigmoid(gram)


def _gcn_run(xT, xsT, fc1_w, fc1_b, fc2_w, fc2_b, gws, gbs,
             idx0, idx1, idx2, n1, n2):
    b, n0, T = xT.shape
    nf = n0 // 4
    full = lambda *shape: pl.BlockSpec(shape, lambda i: tuple(0 for _ in shape))
    bat = lambda *shape: pl.BlockSpec((1,) + shape,
                                      lambda i: (i,) + tuple(0 for _ in shape))
    args = (
        xT, xsT,
        fc1_w, fc1_b.reshape(1, _HID),
        jnp.zeros((128, _HID), jnp.float32).at[0].set(fc2_w[0]),
        jnp.zeros((1, 128), jnp.float32).at[0, 0].set(fc2_b[0]),
        gws[0], gbs[0].reshape(1, _HID),
        gws[1], gbs[1].reshape(1, _HID),
        gws[2], gbs[2].reshape(1, _HID),
        idx0, idx1, idx2, n1, n2,
    )
    in_specs = [
        bat(n0, T), bat(n0, T),
        full(_HID, 4), full(1, _HID), full(128, _HID), full(1, 128),
        full(_HID, _HID), full(1, _HID),
        full(_HID, _HID), full(1, _HID),
        full(_HID, _HID), full(1, _HID),
        bat(n0, 8), bat(n0 // 2, 4), bat(nf, 2),
        bat(n0 // 2, 1), bat(nf, 1),
    ]
    return pl.pallas_call(
        functools.partial(_gcn_body, n0=n0, T=T),
        grid=(b,),
        in_specs=in_specs,
        out_specs=pl.BlockSpec((1, nf, nf), lambda i: (i, 0, 0)),
        out_shape=jax.ShapeDtypeStruct((b, nf, nf), jnp.float32),
    )(*args)


@functools.lru_cache(maxsize=None)
def _gumbel_noise(b, n0):
    with jax.ensure_compile_time_eval():
        key = jax.random.key(42)
        key, k1 = jax.random.split(key)
        ge0 = jax.random.gumbel(k1, (b, n0, n0), jnp.float32)
        key, k2 = jax.random.split(key)
        gn0 = jax.random.gumbel(k2, (b, n0), jnp.float32)
        key, k1 = jax.random.split(key)
        ge1 = jax.random.gumbel(k1, (b, n0 // 2, n0 // 2), jnp.float32)
        key, k2 = jax.random.split(key)
        gn1 = jax.random.gumbel(k2, (b, n0 // 2), jnp.float32)
        key, k1 = jax.random.split(key)
        ge2 = jax.random.gumbel(k1, (b, n0 // 4, n0 // 4), jnp.float32)
    return ge0, gn0, ge1, gn1, ge2


def kernel(x, edge_score, node_score, fc1_w, fc1_b, fc2_w, fc2_b,
           g0_w, g0_b, g1_w, g1_b, g2_w, g2_b):
    b, T, n0 = x.shape
    ge0, gn0, ge1, gn1, ge2 = _gumbel_noise(b, n0)

    vn0 = jax.nn.log_softmax(-node_score, axis=-1) + gn0
    n1 = _node_order(vn0, n0 // 2)
    n1f = n1[..., 0]
    ns1 = jnp.take_along_axis(node_score, n1f, axis=1)
    es1 = jnp.take_along_axis(edge_score, n1f[:, :, None], axis=1)
    es1 = jnp.take_along_axis(es1, n1f[:, None, :], axis=2)
    vn1 = jax.nn.log_softmax(-ns1, axis=-1) + gn1
    n2 = _node_order(vn1, n0 // 4)
    n2f = n2[..., 0]
    es2 = jnp.take_along_axis(es1, n2f[:, :, None], axis=1)
    es2 = jnp.take_along_axis(es2, n2f[:, None, :], axis=2)
    selected = jnp.take_along_axis(n1f, n2f, axis=1)

    idx0 = _edge_topk(edge_score, ge0, 8)
    idx1 = _edge_topk(es1, ge1, 4)
    idx2 = _edge_topk(es2, ge2, 2)

    xs = jnp.argsort(jnp.argsort(x, axis=-1), axis=-1).astype(jnp.float32)
    xT = jnp.swapaxes(x, 1, 2)
    xsT = jnp.swapaxes(xs, 1, 2)

    out = _gcn_run(xT, xsT, fc1_w, fc1_b, fc2_w, fc2_b,
                   [g0_w, g1_w, g2_w], [g0_b, g1_b, g2_b],
                   idx0, idx1, idx2, n1, n2)
    return out, selected
